# baseline (speedup 1.0000x reference)
import functools

import jax
import jax.numpy as jnp
from jax import lax
from jax.experimental import pallas as pl
from jax.experimental.pallas import tpu as pltpu
from jax.experimental.pallas import tpu_sc as plsc

_CHUNK = 2560
_VW = 512


@functools.lru_cache(maxsize=None)
def _make_transpose(vocab: int, d: int):
    info = plsc.get_sparse_core_info()
    nc, ns = info.num_cores, info.num_subcores
    nw = nc * ns

    n_full = vocab // _VW
    tail = vocab - n_full * _VW
    assert d == 16 and tail % 8 == 0
    vocab_pad = n_full * _VW + (128 if tail else 0)

    mesh = plsc.VectorSubcoreMesh(core_axis_name="c", subcore_axis_name="s")

    @functools.partial(
        pl.kernel,
        mesh=mesh,
        compiler_params=pltpu.CompilerParams(use_tc_tiling_on_sc=True,
                                             needs_layout_passes=False),
        out_type=jax.ShapeDtypeStruct((vocab_pad * d,), jnp.float32),
        scratch_types=[
            pltpu.VMEM((d, _VW), jnp.float32),
            pltpu.VMEM((_VW * d,), jnp.float32),
        ],
    )
    def transpose_kernel(tbl_t, out, in_v, out_v):
        wid = lax.axis_index("s") * nc + lax.axis_index("c")
        scaled_iota = lax.iota(jnp.int32, 16) * d

        def do_panel(v0, width):
            pltpu.sync_copy(tbl_t.at[:, pl.ds(v0, width)],
                            in_v.at[:, pl.ds(0, width)])

            def col_block(k, carry):
                base = k * (16 * d)
                for dd in range(d):
                    vals = in_v[dd, pl.ds(k * 16, 16)]
                    plsc.store_scatter(out_v, [scaled_iota + (base + dd)],
                                       vals)
                return carry

            lax.fori_loop(0, width // 16, col_block, 0)
            pltpu.sync_copy(out_v.at[pl.ds(0, width * d)],
                            out.at[pl.ds(v0 * d, width * d)])

        base = n_full // nw
        extra = n_full - base * nw

        def panel_loop(k, carry):
            do_panel((wid + k * nw) * _VW, _VW)
            return carry

        lax.fori_loop(0, base, panel_loop, 0)

        @pl.when(wid < extra)
        def _():
            do_panel((wid + base * nw) * _VW, _VW)


    return transpose_kernel, vocab_pad


_SCH = 10
_BW = 128


@functools.lru_cache(maxsize=None)
def _make_gather(qs: int, cs: int, batch: int, vocab: int, d: int):
    info = plsc.get_sparse_core_info()
    nc, ns = info.num_cores, info.num_subcores
    nw = nc * ns

    assert d == 16 and batch == _BW * nw
    assert qs % _SCH == 0 and cs % _SCH == 0
    rows = _SCH * _BW

    mesh = plsc.VectorSubcoreMesh(core_axis_name="c", subcore_axis_name="s")

    @functools.partial(
        pl.kernel,
        mesh=mesh,
        compiler_params=pltpu.CompilerParams(use_tc_tiling_on_sc=False,
                                             needs_layout_passes=False),
        out_type=(
            jax.ShapeDtypeStruct((qs * 2 * nw * 8 * _BW,), jnp.float32),
            jax.ShapeDtypeStruct((cs * 2 * nw * 8 * _BW,), jnp.float32),
        ),
        scratch_types=[
            pltpu.VMEM((_SCH, _BW), jnp.int32),
            pltpu.VMEM((_SCH, _BW), jnp.int32),
            pltpu.VMEM((rows, d), jnp.float32),
            pltpu.VMEM((rows, d), jnp.float32),
            pltpu.VMEM((_SCH * d * _BW,), jnp.float32),
            pltpu.VMEM((_SCH * d * _BW,), jnp.float32),
            pltpu.SemaphoreType.DMA,
            pltpu.SemaphoreType.DMA,
            pltpu.SemaphoreType.DMA,
            pltpu.SemaphoreType.DMA,
        ],
    )
    def gather_kernel(table, q_idx, c_idx, q_out, c_out,
                      idx_v0, idx_v1, rows_v0, rows_v1, out_v0, out_v1,
                      sem0, sem1, sem_out0, sem_out1):
        wid = lax.axis_index("s") * nc + lax.axis_index("c")
        idx_bufs = (idx_v0, idx_v1)
        row_bufs = (rows_v0, rows_v1)
        out_bufs = (out_v0, out_v1)
        sems = (sem0, sem1)
        out_sems = (sem_out0, sem_out1)
        iota = lax.iota(jnp.int32, 16)
        lane_off = (iota // 8) * (8 * _BW) + (iota % 8) * _BW

        tasks = []
        for j in range(qs // _SCH):
            tasks.append((q_idx, j * _SCH, q_out))
        for j in range(cs // _SCH):
            tasks.append((c_idx, j * _SCH, c_out))

        def start(i):
            src, s0, _ = tasks[i]
            b = i % 2
            pltpu.sync_copy(src.at[pl.ds(s0, _SCH), pl.ds(wid * _BW, _BW)],
                            idx_bufs[b])
            return [pltpu.async_copy(table.at[idx_bufs[b].at[si]],
                                     row_bufs[b].at[pl.ds(si * _BW, _BW)],
                                     sems[b])
                    for si in range(_SCH)]

        def transpose_chunk(rbuf, obuf):
            def outer(si, carry):
                base = lane_off + si * (d * _BW)
                row0 = si * _BW

                def inner(b16, carry):
                    bl = b16 * 16
                    for u in range(16):
                        vals = rbuf[row0 + bl + u, :]
                        plsc.store_scatter(obuf, [base + (bl + u)], vals)
                    return carry

                return lax.fori_loop(0, _BW // 16, inner, carry)

            lax.fori_loop(0, _SCH, outer, 0)

        def write_out(i):
            _, s0, out = tasks[i]
            obuf, osem = out_bufs[i % 2], out_sems[i % 2]
            handles = []
            for si in range(_SCH):
                for r in range(2):
                    src = obuf.at[pl.ds((si * 2 + r) * (8 * _BW), 8 * _BW)]
                    off = (((s0 + si) * 2 + r) * nw + wid) * (8 * _BW)
                    handles.append(
                        pltpu.async_copy(src, out.at[pl.ds(off, 8 * _BW)],
                                         osem))
            return handles

        pending = start(0)
        out_pending = [[], []]
        for i in range(len(tasks)):
            nxt = start(i + 1) if i + 1 < len(tasks) else None
            for h in pending:
                h.wait()
            for h in out_pending[i % 2]:
                h.wait()
            transpose_chunk(row_bufs[i % 2], out_bufs[i % 2])
            out_pending[i % 2] = write_out(i)
            pending = nxt
        for hs in out_pending:
            for h in hs:
                h.wait()

    return gather_kernel


@jax.jit
def _impl(question, context, table):
    vocab, d = table.shape
    batch, qs = question.shape
    _, cs = context.shape
    q_t = question.T.astype(jnp.int32)
    c_t = context.T.astype(jnp.int32)
    transpose, vocab_pad = _make_transpose(vocab, d)
    vmaj_flat = transpose(table.T)
    aligned = (vocab // _VW) * _VW
    if aligned < vocab:
        tail_rows = table[aligned:, :].reshape(-1).astype(jnp.float32)
        vmaj_flat = lax.dynamic_update_slice(vmaj_flat, tail_rows,
                                             (aligned * d,))
    vmaj = vmaj_flat.reshape(vocab_pad, d)
    gather = _make_gather(qs, cs, batch, vocab_pad, d)
    q_raw, c_raw = gather(vmaj, q_t, c_t)
    nw = q_raw.shape[0] // (qs * 2 * 8 * 128)
    q_emb = (q_raw.reshape(qs, 2, nw, 8, 128)
             .transpose(2, 4, 0, 1, 3).reshape(batch, qs, d))
    c_emb = (c_raw.reshape(cs, 2, nw, 8, 128)
             .transpose(2, 4, 0, 1, 3).reshape(batch, cs, d))
    return q_emb, c_emb


def kernel(question, context, table):
    return _impl(question, context, table)


# speedup vs baseline: 1.0065x; 1.0065x over previous
import functools

import jax
import jax.numpy as jnp
from jax import lax
from jax.experimental import pallas as pl
from jax.experimental.pallas import tpu as pltpu
from jax.experimental.pallas import tpu_sc as plsc

_CHUNK = 2560
_VW = 512


@functools.lru_cache(maxsize=None)
def _make_transpose(vocab: int, d: int):
    info = plsc.get_sparse_core_info()
    nc, ns = info.num_cores, info.num_subcores
    nw = nc * ns

    n_full = vocab // _VW
    tail = vocab - n_full * _VW
    assert d == 16 and tail % 8 == 0
    vocab_pad = n_full * _VW + (128 if tail else 0)

    mesh = plsc.VectorSubcoreMesh(core_axis_name="c", subcore_axis_name="s")

    @functools.partial(
        pl.kernel,
        mesh=mesh,
        compiler_params=pltpu.CompilerParams(use_tc_tiling_on_sc=True,
                                             needs_layout_passes=False),
        out_type=jax.ShapeDtypeStruct((vocab_pad * d,), jnp.float32),
        scratch_types=[
            pltpu.VMEM((d, _VW), jnp.float32),
            pltpu.VMEM((_VW * d,), jnp.float32),
        ],
    )
    def transpose_kernel(tbl_t, out, in_v, out_v):
        wid = lax.axis_index("s") * nc + lax.axis_index("c")
        scaled_iota = lax.iota(jnp.int32, 16) * d

        def do_panel(v0, width):
            pltpu.sync_copy(tbl_t.at[:, pl.ds(v0, width)],
                            in_v.at[:, pl.ds(0, width)])

            def col_block(k, carry):
                base = k * (16 * d)
                for dd in range(d):
                    vals = in_v[dd, pl.ds(k * 16, 16)]
                    plsc.store_scatter(out_v, [scaled_iota + (base + dd)],
                                       vals)
                return carry

            lax.fori_loop(0, width // 16, col_block, 0)
            pltpu.sync_copy(out_v.at[pl.ds(0, width * d)],
                            out.at[pl.ds(v0 * d, width * d)])

        base = n_full // nw
        extra = n_full - base * nw

        def panel_loop(k, carry):
            do_panel((wid + k * nw) * _VW, _VW)
            return carry

        lax.fori_loop(0, base, panel_loop, 0)

        @pl.when(wid < extra)
        def _():
            do_panel((wid + base * nw) * _VW, _VW)


    return transpose_kernel, vocab_pad


_SCH = 10
_BW = 128


@functools.lru_cache(maxsize=None)
def _make_gather(qs: int, cs: int, batch: int, vocab: int, d: int):
    info = plsc.get_sparse_core_info()
    nc, ns = info.num_cores, info.num_subcores
    nw = nc * ns

    assert d == 16 and batch == _BW * nw
    assert qs % _SCH == 0 and cs % _SCH == 0
    rows = _SCH * _BW

    mesh = plsc.VectorSubcoreMesh(core_axis_name="c", subcore_axis_name="s")

    @functools.partial(
        pl.kernel,
        mesh=mesh,
        compiler_params=pltpu.CompilerParams(use_tc_tiling_on_sc=False,
                                             needs_layout_passes=False),
        out_type=(
            jax.ShapeDtypeStruct((qs * 2 * nw * 8 * _BW,), jnp.float32),
            jax.ShapeDtypeStruct((cs * 2 * nw * 8 * _BW,), jnp.float32),
        ),
        scratch_types=[
            pltpu.VMEM((rows,), jnp.int32),
            pltpu.VMEM((rows,), jnp.int32),
            pltpu.VMEM((rows, d), jnp.float32),
            pltpu.VMEM((rows, d), jnp.float32),
            pltpu.VMEM((_SCH * d * _BW,), jnp.float32),
            pltpu.VMEM((_SCH * d * _BW,), jnp.float32),
            pltpu.SemaphoreType.DMA,
            pltpu.SemaphoreType.DMA,
            pltpu.SemaphoreType.DMA,
            pltpu.SemaphoreType.DMA,
        ],
    )
    def gather_kernel(table, q_idx, c_idx, q_out, c_out,
                      idx_v0, idx_v1, rows_v0, rows_v1, out_v0, out_v1,
                      sem0, sem1, sem_out0, sem_out1):
        wid = lax.axis_index("s") * nc + lax.axis_index("c")
        idx_bufs = (idx_v0, idx_v1)
        row_bufs = (rows_v0, rows_v1)
        out_bufs = (out_v0, out_v1)
        sems = (sem0, sem1)
        out_sems = (sem_out0, sem_out1)
        iota = lax.iota(jnp.int32, 16)
        lane_off = (iota // 8) * (8 * _BW) + (iota % 8) * _BW

        tasks = []
        for j in range(qs // _SCH):
            tasks.append((q_idx, j * _SCH, q_out))
        for j in range(cs // _SCH):
            tasks.append((c_idx, j * _SCH, c_out))

        def start(i):
            src, s0, _ = tasks[i]
            b = i % 2
            j = s0 // _SCH
            pltpu.sync_copy(src.at[pl.ds((j * nw + wid) * rows, rows)],
                            idx_bufs[b])
            return [pltpu.async_copy(table.at[idx_bufs[b]],
                                     row_bufs[b], sems[b])]

        def transpose_chunk(rbuf, obuf):
            def outer(si, carry):
                base = lane_off + si * (d * _BW)
                row0 = si * _BW

                def inner(b16, carry):
                    bl = b16 * 16
                    for u in range(16):
                        vals = rbuf[row0 + bl + u, :]
                        plsc.store_scatter(obuf, [base + (bl + u)], vals)
                    return carry

                return lax.fori_loop(0, _BW // 16, inner, carry)

            lax.fori_loop(0, _SCH, outer, 0)

        def write_out(i):
            _, s0, out = tasks[i]
            obuf, osem = out_bufs[i % 2], out_sems[i % 2]
            handles = []
            for si in range(_SCH):
                for r in range(2):
                    src = obuf.at[pl.ds((si * 2 + r) * (8 * _BW), 8 * _BW)]
                    off = (((s0 + si) * 2 + r) * nw + wid) * (8 * _BW)
                    handles.append(
                        pltpu.async_copy(src, out.at[pl.ds(off, 8 * _BW)],
                                         osem))
            return handles

        pending = start(0)
        out_pending = [[], []]
        for i in range(len(tasks)):
            nxt = start(i + 1) if i + 1 < len(tasks) else None
            for h in pending:
                h.wait()
            for h in out_pending[i % 2]:
                h.wait()
            transpose_chunk(row_bufs[i % 2], out_bufs[i % 2])
            out_pending[i % 2] = write_out(i)
            pending = nxt
        for hs in out_pending:
            for h in hs:
                h.wait()

    return gather_kernel


@jax.jit
def _impl(question, context, table):
    vocab, d = table.shape
    batch, qs = question.shape
    _, cs = context.shape
    nw0 = batch // _BW
    q_t = (question.T.astype(jnp.int32)
           .reshape(qs // _SCH, _SCH, nw0, _BW)
           .transpose(0, 2, 1, 3).reshape(-1))
    c_t = (context.T.astype(jnp.int32)
           .reshape(cs // _SCH, _SCH, nw0, _BW)
           .transpose(0, 2, 1, 3).reshape(-1))
    transpose, vocab_pad = _make_transpose(vocab, d)
    vmaj_flat = transpose(table.T)
    aligned = (vocab // _VW) * _VW
    if aligned < vocab:
        tail_rows = table[aligned:, :].reshape(-1).astype(jnp.float32)
        vmaj_flat = lax.dynamic_update_slice(vmaj_flat, tail_rows,
                                             (aligned * d,))
    vmaj = vmaj_flat.reshape(vocab_pad, d)
    gather = _make_gather(qs, cs, batch, vocab_pad, d)
    q_raw, c_raw = gather(vmaj, q_t, c_t)
    nw = q_raw.shape[0] // (qs * 2 * 8 * 128)
    q_emb = (q_raw.reshape(qs, 2, nw, 8, 128)
             .transpose(2, 4, 0, 1, 3).reshape(batch, qs, d))
    c_emb = (c_raw.reshape(cs, 2, nw, 8, 128)
             .transpose(2, 4, 0, 1, 3).reshape(batch, cs, d))
    return q_emb, c_emb


def kernel(question, context, table):
    return _impl(question, context, table)


# speedup vs baseline: 1.3484x; 1.3397x over previous
import functools

import jax
import jax.numpy as jnp
from jax import lax
from jax.experimental import pallas as pl
from jax.experimental.pallas import tpu as pltpu
from jax.experimental.pallas import tpu_sc as plsc

_CHUNK = 2560
_VW = 512


@functools.lru_cache(maxsize=None)
def _make_transpose(vocab: int, d: int):
    info = plsc.get_sparse_core_info()
    nc, ns = info.num_cores, info.num_subcores
    nw = nc * ns

    n_full = vocab // _VW
    tail = vocab - n_full * _VW
    assert d == 16 and tail % 8 == 0
    vocab_pad = n_full * _VW + (128 if tail else 0)

    mesh = plsc.VectorSubcoreMesh(core_axis_name="c", subcore_axis_name="s")

    @functools.partial(
        pl.kernel,
        mesh=mesh,
        compiler_params=pltpu.CompilerParams(use_tc_tiling_on_sc=True,
                                             needs_layout_passes=False),
        out_type=jax.ShapeDtypeStruct((vocab_pad * d,), jnp.float32),
        scratch_types=[
            pltpu.VMEM((d, _VW), jnp.float32),
            pltpu.VMEM((_VW * d,), jnp.float32),
        ],
    )
    def transpose_kernel(tbl_t, out, in_v, out_v):
        wid = lax.axis_index("s") * nc + lax.axis_index("c")
        scaled_iota = lax.iota(jnp.int32, 16) * d

        def do_panel(v0, width):
            pltpu.sync_copy(tbl_t.at[:, pl.ds(v0, width)],
                            in_v.at[:, pl.ds(0, width)])

            def col_block(k, carry):
                base = k * (16 * d)
                vals = [in_v[dd, pl.ds(k * 16, 16)] for dd in range(d)]
                for dd in range(d):
                    plsc.store_scatter(out_v, [scaled_iota + (base + dd)],
                                       vals[dd])
                return carry

            lax.fori_loop(0, width // 16, col_block, 0)
            pltpu.sync_copy(out_v.at[pl.ds(0, width * d)],
                            out.at[pl.ds(v0 * d, width * d)])

        base = n_full // nw
        extra = n_full - base * nw

        def panel_loop(k, carry):
            do_panel((wid + k * nw) * _VW, _VW)
            return carry

        lax.fori_loop(0, base, panel_loop, 0)

        @pl.when(wid < extra)
        def _():
            do_panel((wid + base * nw) * _VW, _VW)


    return transpose_kernel, vocab_pad


_SCH = 10
_BW = 128


@functools.lru_cache(maxsize=None)
def _make_gather(qs: int, cs: int, batch: int, vocab: int, d: int):
    info = plsc.get_sparse_core_info()
    nc, ns = info.num_cores, info.num_subcores
    nw = nc * ns

    assert d == 16 and batch == _BW * nw
    assert qs % _SCH == 0 and cs % _SCH == 0
    rows = _SCH * _BW

    mesh = plsc.VectorSubcoreMesh(core_axis_name="c", subcore_axis_name="s")

    @functools.partial(
        pl.kernel,
        mesh=mesh,
        compiler_params=pltpu.CompilerParams(use_tc_tiling_on_sc=False,
                                             needs_layout_passes=False),
        out_type=(
            jax.ShapeDtypeStruct((qs * 2 * nw * 8 * _BW,), jnp.float32),
            jax.ShapeDtypeStruct((cs * 2 * nw * 8 * _BW,), jnp.float32),
        ),
        scratch_types=[
            pltpu.VMEM((rows,), jnp.int32),
            pltpu.VMEM((rows,), jnp.int32),
            pltpu.VMEM((rows, d), jnp.float32),
            pltpu.VMEM((rows, d), jnp.float32),
            pltpu.VMEM((_SCH * d * _BW,), jnp.float32),
            pltpu.VMEM((_SCH * d * _BW,), jnp.float32),
            pltpu.SemaphoreType.DMA,
            pltpu.SemaphoreType.DMA,
            pltpu.SemaphoreType.DMA,
            pltpu.SemaphoreType.DMA,
        ],
    )
    def gather_kernel(table, q_idx, c_idx, q_out, c_out,
                      idx_v0, idx_v1, rows_v0, rows_v1, out_v0, out_v1,
                      sem0, sem1, sem_out0, sem_out1):
        wid = lax.axis_index("s") * nc + lax.axis_index("c")
        idx_bufs = (idx_v0, idx_v1)
        row_bufs = (rows_v0, rows_v1)
        out_bufs = (out_v0, out_v1)
        sems = (sem0, sem1)
        out_sems = (sem_out0, sem_out1)
        iota = lax.iota(jnp.int32, 16)
        lane_off = (iota // 8) * (8 * _BW) + (iota % 8) * _BW

        tasks = []
        for j in range(qs // _SCH):
            tasks.append((q_idx, j * _SCH, q_out))
        for j in range(cs // _SCH):
            tasks.append((c_idx, j * _SCH, c_out))

        def start(i):
            src, s0, _ = tasks[i]
            b = i % 2
            j = s0 // _SCH
            pltpu.sync_copy(src.at[pl.ds((j * nw + wid) * rows, rows)],
                            idx_bufs[b])
            return [pltpu.async_copy(table.at[idx_bufs[b]],
                                     row_bufs[b], sems[b])]

        def transpose_chunk(rbuf, obuf):
            def outer(si, carry):
                base = lane_off + si * (d * _BW)
                row0 = si * _BW

                def inner(b16, carry):
                    bl = b16 * 16
                    vals = [rbuf[row0 + bl + u, :] for u in range(16)]
                    for u in range(16):
                        plsc.store_scatter(obuf, [base + (bl + u)], vals[u])
                    return carry

                return lax.fori_loop(0, _BW // 16, inner, carry)

            lax.fori_loop(0, _SCH, outer, 0)

        def write_out(i):
            _, s0, out = tasks[i]
            obuf, osem = out_bufs[i % 2], out_sems[i % 2]
            handles = []
            for si in range(_SCH):
                for r in range(2):
                    src = obuf.at[pl.ds((si * 2 + r) * (8 * _BW), 8 * _BW)]
                    off = (((s0 + si) * 2 + r) * nw + wid) * (8 * _BW)
                    handles.append(
                        pltpu.async_copy(src, out.at[pl.ds(off, 8 * _BW)],
                                         osem))
            return handles

        pending = start(0)
        out_pending = [[], []]
        for i in range(len(tasks)):
            nxt = start(i + 1) if i + 1 < len(tasks) else None
            for h in pending:
                h.wait()
            for h in out_pending[i % 2]:
                h.wait()
            transpose_chunk(row_bufs[i % 2], out_bufs[i % 2])
            out_pending[i % 2] = write_out(i)
            pending = nxt
        for hs in out_pending:
            for h in hs:
                h.wait()

    return gather_kernel


@jax.jit
def _impl(question, context, table):
    vocab, d = table.shape
    batch, qs = question.shape
    _, cs = context.shape
    nw0 = batch // _BW
    q_t = (question.T.astype(jnp.int32)
           .reshape(qs // _SCH, _SCH, nw0, _BW)
           .transpose(0, 2, 1, 3).reshape(-1))
    c_t = (context.T.astype(jnp.int32)
           .reshape(cs // _SCH, _SCH, nw0, _BW)
           .transpose(0, 2, 1, 3).reshape(-1))
    transpose, vocab_pad = _make_transpose(vocab, d)
    vmaj_flat = transpose(table.T)
    aligned = (vocab // _VW) * _VW
    if aligned < vocab:
        tail_rows = table[aligned:, :].reshape(-1).astype(jnp.float32)
        vmaj_flat = lax.dynamic_update_slice(vmaj_flat, tail_rows,
                                             (aligned * d,))
    vmaj = vmaj_flat.reshape(vocab_pad, d)
    gather = _make_gather(qs, cs, batch, vocab_pad, d)
    q_raw, c_raw = gather(vmaj, q_t, c_t)
    nw = q_raw.shape[0] // (qs * 2 * 8 * 128)
    q_emb = (q_raw.reshape(qs, 2, nw, 8, 128)
             .transpose(2, 4, 0, 1, 3).reshape(batch, qs, d))
    c_emb = (c_raw.reshape(cs, 2, nw, 8, 128)
             .transpose(2, 4, 0, 1, 3).reshape(batch, cs, d))
    return q_emb, c_emb


def kernel(question, context, table):
    return _impl(question, context, table)


# speedup vs baseline: 1.3489x; 1.0004x over previous
import functools

import jax
import jax.numpy as jnp
from jax import lax
from jax.experimental import pallas as pl
from jax.experimental.pallas import tpu as pltpu
from jax.experimental.pallas import tpu_sc as plsc

_CHUNK = 2560
_VW = 512


@functools.lru_cache(maxsize=None)
def _make_transpose(vocab: int, d: int):
    info = plsc.get_sparse_core_info()
    nc, ns = info.num_cores, info.num_subcores
    nw = nc * ns

    n_full = vocab // _VW
    tail = vocab - n_full * _VW
    assert d == 16 and tail % 8 == 0
    vocab_pad = n_full * _VW + (128 if tail else 0)

    mesh = plsc.VectorSubcoreMesh(core_axis_name="c", subcore_axis_name="s")

    @functools.partial(
        pl.kernel,
        mesh=mesh,
        compiler_params=pltpu.CompilerParams(use_tc_tiling_on_sc=True,
                                             needs_layout_passes=False),
        out_type=jax.ShapeDtypeStruct((vocab_pad * d,), jnp.float32),
        scratch_types=[
            pltpu.VMEM((d, _VW), jnp.float32),
            pltpu.VMEM((_VW * d,), jnp.float32),
        ],
    )
    def transpose_kernel(tbl_t, out, in_v, out_v):
        wid = lax.axis_index("s") * nc + lax.axis_index("c")
        scaled_iota = lax.iota(jnp.int32, 16) * d

        def do_panel(v0, width):
            pltpu.sync_copy(tbl_t.at[:, pl.ds(v0, width)],
                            in_v.at[:, pl.ds(0, width)])

            def col_block(k, carry):
                base = k * (16 * d)
                vals = [in_v[dd, pl.ds(k * 16, 16)] for dd in range(d)]
                for dd in range(d):
                    plsc.store_scatter(out_v, [scaled_iota + (base + dd)],
                                       vals[dd])
                return carry

            lax.fori_loop(0, width // 16, col_block, 0)
            pltpu.sync_copy(out_v.at[pl.ds(0, width * d)],
                            out.at[pl.ds(v0 * d, width * d)])

        base = n_full // nw
        extra = n_full - base * nw

        def panel_loop(k, carry):
            do_panel((wid + k * nw) * _VW, _VW)
            return carry

        lax.fori_loop(0, base, panel_loop, 0)

        @pl.when(wid < extra)
        def _():
            do_panel((wid + base * nw) * _VW, _VW)


    return transpose_kernel, vocab_pad


_SCH = 10
_BW = 128


@functools.lru_cache(maxsize=None)
def _make_gather(qs: int, cs: int, batch: int, vocab: int, d: int):
    info = plsc.get_sparse_core_info()
    nc, ns = info.num_cores, info.num_subcores
    nw = nc * ns

    assert d == 16 and batch == _BW * nw
    assert qs % _SCH == 0 and cs % _SCH == 0
    rows = _SCH * _BW

    mesh = plsc.VectorSubcoreMesh(core_axis_name="c", subcore_axis_name="s")

    @functools.partial(
        pl.kernel,
        mesh=mesh,
        compiler_params=pltpu.CompilerParams(use_tc_tiling_on_sc=False,
                                             needs_layout_passes=False),
        out_type=(
            jax.ShapeDtypeStruct((qs * 2 * nw * 8, _BW), jnp.float32),
            jax.ShapeDtypeStruct((cs * 2 * nw * 8, _BW), jnp.float32),
        ),
        scratch_types=[
            pltpu.VMEM((rows,), jnp.int32),
            pltpu.VMEM((rows,), jnp.int32),
            pltpu.VMEM((rows, d), jnp.float32),
            pltpu.VMEM((rows, d), jnp.float32),
            pltpu.VMEM((_SCH * 2 * 8, _BW), jnp.float32),
            pltpu.VMEM((_SCH * 2 * 8, _BW), jnp.float32),
            pltpu.SemaphoreType.DMA,
            pltpu.SemaphoreType.DMA,
            pltpu.SemaphoreType.DMA,
            pltpu.SemaphoreType.DMA,
        ],
    )
    def gather_kernel(table, q_idx, c_idx, q_out, c_out,
                      idx_v0, idx_v1, rows_v0, rows_v1, out_v0, out_v1,
                      sem0, sem1, sem_out0, sem_out1):
        wid = lax.axis_index("s") * nc + lax.axis_index("c")
        idx_bufs = (idx_v0, idx_v1)
        row_bufs = (rows_v0, rows_v1)
        out_bufs = (out_v0, out_v1)
        sems = (sem0, sem1)
        out_sems = (sem_out0, sem_out1)
        iota = lax.iota(jnp.int32, 16)
        lane_hi = iota // 8
        lane_lo = iota % 8

        tasks = []
        for j in range(qs // _SCH):
            tasks.append((q_idx, j * _SCH, q_out))
        for j in range(cs // _SCH):
            tasks.append((c_idx, j * _SCH, c_out))

        def start(i):
            src, s0, _ = tasks[i]
            b = i % 2
            j = s0 // _SCH
            pltpu.sync_copy(src.at[pl.ds((j * nw + wid) * rows, rows)],
                            idx_bufs[b])
            return [pltpu.async_copy(table.at[idx_bufs[b]],
                                     row_bufs[b], sems[b])]

        def transpose_chunk(rbuf, obuf):
            def outer(si, carry):
                row_vec = iota + si * 16
                row0 = si * _BW

                def inner(b16, carry):
                    bl = b16 * 16
                    vals = [rbuf[row0 + bl + u, :] for u in range(16)]
                    for u in range(16):
                        plsc.store_scatter(
                            obuf,
                            [row_vec, jnp.full((16,), bl + u, jnp.int32)],
                            vals[u])
                    return carry

                return lax.fori_loop(0, _BW // 16, inner, carry)

            lax.fori_loop(0, _SCH, outer, 0)

        def write_out(i):
            _, s0, out = tasks[i]
            obuf, osem = out_bufs[i % 2], out_sems[i % 2]
            handles = []
            for si in range(_SCH):
                for r in range(2):
                    src = obuf.at[pl.ds((si * 2 + r) * 8, 8), :]
                    off = (((s0 + si) * 2 + r) * nw + wid) * 8
                    handles.append(
                        pltpu.async_copy(src, out.at[pl.ds(off, 8), :],
                                         osem))
            return handles

        pending = start(0)
        out_pending = [[], []]
        for i in range(len(tasks)):
            nxt = start(i + 1) if i + 1 < len(tasks) else None
            for h in pending:
                h.wait()
            for h in out_pending[i % 2]:
                h.wait()
            transpose_chunk(row_bufs[i % 2], out_bufs[i % 2])
            out_pending[i % 2] = write_out(i)
            pending = nxt
        for hs in out_pending:
            for h in hs:
                h.wait()

    return gather_kernel


@jax.jit
def _impl(question, context, table):
    vocab, d = table.shape
    batch, qs = question.shape
    _, cs = context.shape
    nw0 = batch // _BW
    q_t = (question.T.astype(jnp.int32)
           .reshape(qs // _SCH, _SCH, nw0, _BW)
           .transpose(0, 2, 1, 3).reshape(-1))
    c_t = (context.T.astype(jnp.int32)
           .reshape(cs // _SCH, _SCH, nw0, _BW)
           .transpose(0, 2, 1, 3).reshape(-1))
    transpose, vocab_pad = _make_transpose(vocab, d)
    vmaj_flat = transpose(table.T)
    aligned = (vocab // _VW) * _VW
    if aligned < vocab:
        tail_rows = table[aligned:, :].reshape(-1).astype(jnp.float32)
        vmaj_flat = lax.dynamic_update_slice(vmaj_flat, tail_rows,
                                             (aligned * d,))
    vmaj = vmaj_flat.reshape(vocab_pad, d)
    gather = _make_gather(qs, cs, batch, vocab_pad, d)
    q_raw, c_raw = gather(vmaj, q_t, c_t)
    nw = q_raw.shape[0] // (qs * 2 * 8)
    q_emb = (q_raw.reshape(qs, 2, nw, 8, 128)
             .transpose(2, 4, 0, 1, 3).reshape(batch, qs, d))
    c_emb = (c_raw.reshape(cs, 2, nw, 8, 128)
             .transpose(2, 4, 0, 1, 3).reshape(batch, cs, d))
    return q_emb, c_emb


def kernel(question, context, table):
    return _impl(question, context, table)


# speedup vs baseline: 1.5873x; 1.1768x over previous
import functools

import jax
import jax.numpy as jnp
from jax import lax
from jax.experimental import pallas as pl
from jax.experimental.pallas import tpu as pltpu
from jax.experimental.pallas import tpu_sc as plsc

_CHUNK = 2560
_VW = 512


@functools.lru_cache(maxsize=None)
def _make_transpose(vocab: int, d: int):
    info = plsc.get_sparse_core_info()
    nc, ns = info.num_cores, info.num_subcores
    nw = nc * ns

    n_full = vocab // _VW
    tail = vocab - n_full * _VW
    assert d == 16 and tail % 8 == 0
    vocab_pad = n_full * _VW + (128 if tail else 0)

    mesh = plsc.VectorSubcoreMesh(core_axis_name="c", subcore_axis_name="s")

    @functools.partial(
        pl.kernel,
        mesh=mesh,
        compiler_params=pltpu.CompilerParams(use_tc_tiling_on_sc=True,
                                             needs_layout_passes=False),
        out_type=jax.ShapeDtypeStruct((vocab_pad * d,), jnp.float32),
        scratch_types=[
            pltpu.VMEM((d, _VW), jnp.float32),
            pltpu.VMEM((d, _VW), jnp.float32),
            pltpu.VMEM((_VW * d,), jnp.float32),
            pltpu.SemaphoreType.DMA,
            pltpu.SemaphoreType.DMA,
        ],
    )
    def transpose_kernel(tbl_t, out, in_va, in_vb, out_v, sem_a, sem_b):
        wid = lax.axis_index("s") * nc + lax.axis_index("c")
        scaled_iota = lax.iota(jnp.int32, 16) * d

        base = n_full // nw
        extra = n_full - base * nw

        def panel_v0(k):
            return (wid + k * nw) * _VW

        def fetch(k, buf, sem):
            return pltpu.async_copy(tbl_t.at[:, pl.ds(panel_v0(k), _VW)],
                                    buf, sem)

        def wait_fetch(buf, sem):
            pltpu.make_async_copy(tbl_t.at[:, pl.ds(0, _VW)], buf,
                                  sem).wait()

        def compute_store(v0, buf):
            def col_block(k, carry):
                b2 = k * (16 * d)
                vals = [buf[dd, pl.ds(k * 16, 16)] for dd in range(d)]
                for dd in range(d):
                    plsc.store_scatter(out_v, [scaled_iota + (b2 + dd)],
                                       vals[dd])
                return carry

            lax.fori_loop(0, _VW // 16, col_block, 0)
            pltpu.sync_copy(out_v, out.at[pl.ds(v0 * d, _VW * d)])

        n_pairs = base // 2
        fetch(0, in_va, sem_a)

        def pair_loop(p, carry):
            k0 = 2 * p
            fetch(k0 + 1, in_vb, sem_b)
            wait_fetch(in_va, sem_a)
            compute_store(panel_v0(k0), in_va)
            nxt = jnp.minimum(k0 + 2, base - 1)
            fetch(nxt, in_va, sem_a)
            wait_fetch(in_vb, sem_b)
            compute_store(panel_v0(k0 + 1), in_vb)
            return carry

        lax.fori_loop(0, n_pairs, pair_loop, 0)

        wait_fetch(in_va, sem_a)
        if base % 2:
            compute_store(panel_v0(base - 1), in_va)

        @pl.when(wid < extra)
        def _():
            v0 = (wid + base * nw) * _VW
            pltpu.sync_copy(tbl_t.at[:, pl.ds(v0, _VW)], in_va)
            compute_store(v0, in_va)


    return transpose_kernel, vocab_pad


_SCH = 10
_BW = 128


@functools.lru_cache(maxsize=None)
def _make_gather(qs: int, cs: int, batch: int, vocab: int, d: int):
    info = plsc.get_sparse_core_info()
    nc, ns = info.num_cores, info.num_subcores
    nw = nc * ns

    assert d == 16 and batch == _BW * nw
    assert qs % _SCH == 0 and cs % _SCH == 0
    rows = _SCH * _BW

    mesh = plsc.VectorSubcoreMesh(core_axis_name="c", subcore_axis_name="s")

    @functools.partial(
        pl.kernel,
        mesh=mesh,
        compiler_params=pltpu.CompilerParams(use_tc_tiling_on_sc=False,
                                             needs_layout_passes=False),
        out_type=(
            jax.ShapeDtypeStruct((qs * 2 * nw * 8, _BW), jnp.float32),
            jax.ShapeDtypeStruct((cs * 2 * nw * 8, _BW), jnp.float32),
        ),
        scratch_types=[
            pltpu.VMEM((rows,), jnp.int32),
            pltpu.VMEM((rows,), jnp.int32),
            pltpu.VMEM((rows, d), jnp.float32),
            pltpu.VMEM((rows, d), jnp.float32),
            pltpu.VMEM((_SCH * 2 * 8, _BW), jnp.float32),
            pltpu.VMEM((_SCH * 2 * 8, _BW), jnp.float32),
            pltpu.SemaphoreType.DMA,
            pltpu.SemaphoreType.DMA,
            pltpu.SemaphoreType.DMA,
            pltpu.SemaphoreType.DMA,
        ],
    )
    def gather_kernel(table, q_idx, c_idx, q_out, c_out,
                      idx_v0, idx_v1, rows_v0, rows_v1, out_v0, out_v1,
                      sem0, sem1, sem_out0, sem_out1):
        wid = lax.axis_index("s") * nc + lax.axis_index("c")
        idx_bufs = (idx_v0, idx_v1)
        row_bufs = (rows_v0, rows_v1)
        out_bufs = (out_v0, out_v1)
        sems = (sem0, sem1)
        out_sems = (sem_out0, sem_out1)
        iota = lax.iota(jnp.int32, 16)
        lane_hi = iota // 8
        lane_lo = iota % 8

        tasks = []
        for j in range(qs // _SCH):
            tasks.append((q_idx, j * _SCH, q_out))
        for j in range(cs // _SCH):
            tasks.append((c_idx, j * _SCH, c_out))

        def start(i):
            src, s0, _ = tasks[i]
            b = i % 2
            j = s0 // _SCH
            pltpu.sync_copy(src.at[pl.ds((j * nw + wid) * rows, rows)],
                            idx_bufs[b])
            return [pltpu.async_copy(table.at[idx_bufs[b]],
                                     row_bufs[b], sems[b])]

        def transpose_chunk(rbuf, obuf):
            def outer(si, carry):
                row_vec = iota + si * 16
                row0 = si * _BW

                def inner(b16, carry):
                    bl = b16 * 16
                    vals = [rbuf[row0 + bl + u, :] for u in range(16)]
                    for u in range(16):
                        plsc.store_scatter(
                            obuf,
                            [row_vec, jnp.full((16,), bl + u, jnp.int32)],
                            vals[u])
                    return carry

                return lax.fori_loop(0, _BW // 16, inner, carry)

            lax.fori_loop(0, _SCH, outer, 0)

        def write_out(i):
            _, s0, out = tasks[i]
            obuf, osem = out_bufs[i % 2], out_sems[i % 2]
            handles = []
            for si in range(_SCH):
                for r in range(2):
                    src = obuf.at[pl.ds((si * 2 + r) * 8, 8), :]
                    off = (((s0 + si) * 2 + r) * nw + wid) * 8
                    handles.append(
                        pltpu.async_copy(src, out.at[pl.ds(off, 8), :],
                                         osem))
            return handles

        pending = start(0)
        out_pending = [[], []]
        for i in range(len(tasks)):
            nxt = start(i + 1) if i + 1 < len(tasks) else None
            for h in pending:
                h.wait()
            for h in out_pending[i % 2]:
                h.wait()
            transpose_chunk(row_bufs[i % 2], out_bufs[i % 2])
            out_pending[i % 2] = write_out(i)
            pending = nxt
        for hs in out_pending:
            for h in hs:
                h.wait()

    return gather_kernel


@jax.jit
def _impl(question, context, table):
    vocab, d = table.shape
    batch, qs = question.shape
    _, cs = context.shape
    nw0 = batch // _BW
    q_t = (question.T.astype(jnp.int32)
           .reshape(qs // _SCH, _SCH, nw0, _BW)
           .transpose(0, 2, 1, 3).reshape(-1))
    c_t = (context.T.astype(jnp.int32)
           .reshape(cs // _SCH, _SCH, nw0, _BW)
           .transpose(0, 2, 1, 3).reshape(-1))
    transpose, vocab_pad = _make_transpose(vocab, d)
    vmaj_flat = transpose(table.T)
    aligned = (vocab // _VW) * _VW
    if aligned < vocab:
        tail_rows = table[aligned:, :].reshape(-1).astype(jnp.float32)
        vmaj_flat = lax.dynamic_update_slice(vmaj_flat, tail_rows,
                                             (aligned * d,))
    vmaj = vmaj_flat.reshape(vocab_pad, d)
    gather = _make_gather(qs, cs, batch, vocab_pad, d)
    q_raw, c_raw = gather(vmaj, q_t, c_t)
    nw = q_raw.shape[0] // (qs * 2 * 8)
    q_emb = (q_raw.reshape(qs, 2, nw, 8, 128)
             .transpose(2, 4, 0, 1, 3).reshape(batch, qs, d))
    c_emb = (c_raw.reshape(cs, 2, nw, 8, 128)
             .transpose(2, 4, 0, 1, 3).reshape(batch, cs, d))
    return q_emb, c_emb


def kernel(question, context, table):
    return _impl(question, context, table)
